# baseline (speedup 1.0000x reference)
import functools

import jax
import jax.numpy as jnp
from jax import lax
from jax.experimental import pallas as pl
from jax.experimental.pallas import tpu as pltpu
from jax.experimental.pallas import tpu_sc as plsc

_NW = 32
_SPW = 128
_CHUNK = 128
_K = _CHUNK // _SPW
_NB = 5


@functools.cache
def _make_gather(B, V, D):
    b_per_w = B // _NW
    n_chunks = b_per_w // _CHUNK
    n_q = b_per_w // _SPW
    mesh = plsc.VectorSubcoreMesh(core_axis_name="c", subcore_axis_name="s")

    @functools.partial(
        pl.kernel,
        out_type=jax.ShapeDtypeStruct((B, D), jnp.float32),
        mesh=mesh,
        scratch_types=[
            pltpu.VMEM((n_q, _SPW), jnp.int32),
            [pltpu.VMEM((_CHUNK, D), jnp.float32) for _ in range(_NB)],
            [pltpu.SemaphoreType.DMA for _ in range(_NB)],
            [pltpu.SemaphoreType.DMA for _ in range(_NB)],
        ],
        compiler_params=pltpu.CompilerParams(use_tc_tiling_on_sc=False),
    )
    def gather(table_hbm, idx_hbm, out_hbm, idx_v, rows, semg, semw):
        wid = lax.axis_index("s") * 2 + lax.axis_index("c")
        base = wid * b_per_w
        qbase = wid * n_q

        pltpu.sync_copy(idx_hbm.at[pl.ds(qbase, n_q)], idx_v)

        def fire_gather(g, b):
            for j in range(_K):
                pltpu.async_copy(
                    table_hbm.at[idx_v.at[g * _K + j]],
                    rows[b].at[pl.ds(j * _SPW, _SPW)],
                    semg[b],
                )

        def wait_gather(g, b):
            for j in range(_K):
                pltpu.make_async_copy(
                    table_hbm.at[idx_v.at[g * _K + j]],
                    rows[b].at[pl.ds(j * _SPW, _SPW)],
                    semg[b],
                ).wait()

        def fire_write(g, b):
            pltpu.async_copy(
                rows[b], out_hbm.at[pl.ds(base + g * _CHUNK, _CHUNK)], semw[b]
            )

        def wait_write(g, b):
            pltpu.make_async_copy(
                rows[b], out_hbm.at[pl.ds(base + g * _CHUNK, _CHUNK)], semw[b]
            ).wait()

        for b in range(_NB):
            fire_gather(b, b)

        def body(t, carry):
            g0 = t * _NB
            for b in range(_NB):
                g = g0 + b
                wait_gather(g, b)
                fire_write(g, b)
            for b in range(_NB):
                g = g0 + b
                wait_write(g, b)
                fire_gather(g + _NB, b)
            return carry

        lax.fori_loop(0, n_chunks // _NB - 1, body, 0)
        g0 = n_chunks - _NB
        for b in range(_NB):
            wait_gather(g0 + b, b)
            fire_write(g0 + b, b)
        for b in range(_NB):
            wait_write(g0 + b, b)

    return gather


def kernel(x, table):
    Bm, T = x.shape
    V, D = table.shape
    B = Bm * T
    idx2 = x.reshape(B // _SPW, _SPW).astype(jnp.int32)
    table_pad = jnp.pad(table, ((0, 0), (0, 128 - D)))
    out = _make_gather(B, V, 128)(table_pad, idx2)
    return out[:, :D].reshape(Bm, T, D)


# speedup vs baseline: 1.3969x; 1.3969x over previous
import functools

import jax
import jax.numpy as jnp
from jax import lax
from jax.experimental import pallas as pl
from jax.experimental.pallas import tpu as pltpu
from jax.experimental.pallas import tpu_sc as plsc

_NW = 32
_SPW = 128
_CHUNK = 128
_K = _CHUNK // _SPW
_NB = 5


@functools.cache
def _make_gather(B, V, D):
    b_per_w = B // _NW
    n_chunks = b_per_w // _CHUNK
    n_q = b_per_w // _SPW
    mesh = plsc.VectorSubcoreMesh(core_axis_name="c", subcore_axis_name="s")

    @functools.partial(
        pl.kernel,
        out_type=jax.ShapeDtypeStruct((B, D), jnp.float32),
        mesh=mesh,
        scratch_types=[
            pltpu.VMEM((n_q, _SPW), jnp.int32),
            [pltpu.VMEM((_CHUNK, D), jnp.float32) for _ in range(_NB)],
            [pltpu.SemaphoreType.DMA for _ in range(_NB)],
            [pltpu.SemaphoreType.DMA for _ in range(_NB)],
        ],
        compiler_params=pltpu.CompilerParams(use_tc_tiling_on_sc=False),
    )
    def gather(table_hbm, idx_hbm, out_hbm, idx_v, rows, semg, semw):
        wid = lax.axis_index("s") * 2 + lax.axis_index("c")
        base = wid * b_per_w
        qbase = wid * n_q

        pltpu.sync_copy(idx_hbm.at[pl.ds(qbase, n_q)], idx_v)

        def fire_gather(g, b):
            for j in range(_K):
                pltpu.async_copy(
                    table_hbm.at[idx_v.at[g * _K + j]],
                    rows[b].at[pl.ds(j * _SPW, _SPW)],
                    semg[b],
                )

        def wait_gather(g, b):
            for j in range(_K):
                pltpu.make_async_copy(
                    table_hbm.at[idx_v.at[g * _K + j]],
                    rows[b].at[pl.ds(j * _SPW, _SPW)],
                    semg[b],
                ).wait()

        def fire_write(g, b):
            pltpu.async_copy(
                rows[b], out_hbm.at[pl.ds(base + g * _CHUNK, _CHUNK)], semw[b]
            )

        def wait_write(g, b):
            pltpu.make_async_copy(
                rows[b], out_hbm.at[pl.ds(base + g * _CHUNK, _CHUNK)], semw[b]
            ).wait()

        for b in range(_NB):
            fire_gather(b, b)

        def body(t, carry):
            g0 = t * _NB
            for b in range(_NB):
                g = g0 + b
                wait_gather(g, b)
                fire_write(g, b)
            for b in range(_NB):
                g = g0 + b
                wait_write(g, b)
                fire_gather(g + _NB, b)
            return carry

        lax.fori_loop(0, n_chunks // _NB - 1, body, 0)
        g0 = n_chunks - _NB
        for b in range(_NB):
            wait_gather(g0 + b, b)
            fire_write(g0 + b, b)
        for b in range(_NB):
            wait_write(g0 + b, b)

    return gather


def kernel(x, table):
    Bm, T = x.shape
    V, D = table.shape
    B = Bm * T
    idx2 = x.reshape(B // _SPW, _SPW).astype(jnp.int32)
    eye_pad = jnp.eye(D, 128, dtype=jnp.float32)
    table_pad = jax.lax.dot_general(
        table, eye_pad, (((1,), (0,)), ((), ())),
        preferred_element_type=jnp.float32,
    )
    out = _make_gather(B, V, 128)(table_pad, idx2)
    return out[:, :D].reshape(Bm, T, D)


# speedup vs baseline: 1.4222x; 1.0181x over previous
import functools

import jax
import jax.numpy as jnp
from jax import lax
from jax.experimental import pallas as pl
from jax.experimental.pallas import tpu as pltpu
from jax.experimental.pallas import tpu_sc as plsc

_NW = 32
_SPW = 128
_CHUNK = 128
_K = _CHUNK // _SPW
_NB = 5
_D = 64


@functools.cache
def _make_gather(B, V, D):
    b_per_w = B // _NW
    n_chunks = b_per_w // _CHUNK
    n_q = b_per_w // _SPW
    mesh = plsc.VectorSubcoreMesh(core_axis_name="c", subcore_axis_name="s")

    @functools.partial(
        pl.kernel,
        out_type=jax.ShapeDtypeStruct((B, D), jnp.float32),
        mesh=mesh,
        scratch_types=[
            pltpu.VMEM((n_q, _SPW), jnp.int32),
            [pltpu.VMEM((_CHUNK, D), jnp.float32) for _ in range(_NB)],
            [pltpu.SemaphoreType.DMA for _ in range(_NB)],
            [pltpu.SemaphoreType.DMA for _ in range(_NB)],
        ],
        compiler_params=pltpu.CompilerParams(use_tc_tiling_on_sc=False),
    )
    def gather(table_hbm, idx_hbm, out_hbm, idx_v, rows, semg, semw):
        wid = lax.axis_index("s") * 2 + lax.axis_index("c")
        base = wid * b_per_w
        qbase = wid * n_q

        pltpu.sync_copy(idx_hbm.at[pl.ds(qbase, n_q)], idx_v)

        def fire_gather(g, b):
            for j in range(_K):
                pltpu.async_copy(
                    table_hbm.at[idx_v.at[g * _K + j]],
                    rows[b].at[pl.ds(j * _SPW, _SPW)],
                    semg[b],
                )

        def wait_gather(g, b):
            for j in range(_K):
                pltpu.make_async_copy(
                    table_hbm.at[idx_v.at[g * _K + j]],
                    rows[b].at[pl.ds(j * _SPW, _SPW)],
                    semg[b],
                ).wait()

        def fire_write(g, b):
            pltpu.async_copy(
                rows[b].at[pl.ds(0, _CHUNK), pl.ds(0, _D)],
                out_hbm.at[pl.ds(base + g * _CHUNK, _CHUNK), pl.ds(0, _D)],
                semw[b],
            )

        def wait_write(g, b):
            pltpu.make_async_copy(
                rows[b].at[pl.ds(0, _CHUNK), pl.ds(0, _D)],
                out_hbm.at[pl.ds(base + g * _CHUNK, _CHUNK), pl.ds(0, _D)],
                semw[b],
            ).wait()

        for b in range(_NB):
            fire_gather(b, b)

        def body(t, carry):
            g0 = t * _NB
            for b in range(_NB):
                g = g0 + b
                wait_gather(g, b)
                fire_write(g, b)
            for b in range(_NB):
                g = g0 + b
                wait_write(g, b)
                fire_gather(g + _NB, b)
            return carry

        lax.fori_loop(0, n_chunks // _NB - 1, body, 0)
        g0 = n_chunks - _NB
        for b in range(_NB):
            wait_gather(g0 + b, b)
            fire_write(g0 + b, b)
        for b in range(_NB):
            wait_write(g0 + b, b)

    return gather


def kernel(x, table):
    Bm, T = x.shape
    V, D = table.shape
    B = Bm * T
    idx2 = x.reshape(B // _SPW, _SPW).astype(jnp.int32)
    eye_pad = jnp.eye(D, 128, dtype=jnp.float32)
    table_pad = jax.lax.dot_general(
        table, eye_pad, (((1,), (0,)), ((), ())),
        preferred_element_type=jnp.float32,
    )
    out = _make_gather(B, V, 128)(table_pad, idx2)
    return out[:, :D].reshape(Bm, T, D)
